# Initial kernel scaffold; baseline (speedup 1.0000x reference)
import jax
import jax.numpy as jnp
from jax.experimental import pallas as pl


def kernel(x, z_prev, W, S):
    raise NotImplementedError("write your pallas kernel here")


# speedup vs baseline: 23.4655x; 23.4655x over previous
import functools

import jax
import jax.numpy as jnp
from jax.experimental import pallas as pl

_K = 16
_CODE = 128
_IN = 64


def _body(x_ref, z_ref, w_ref, s_ref, o_ref):
    u = jax.lax.dot_general(
        x_ref[...], w_ref[...], (((1,), (1,)), ((), ())),
        preferred_element_type=jnp.float32)
    u = u + jax.lax.dot_general(
        z_ref[...], s_ref[...], (((1,), (1,)), ((), ())),
        preferred_element_type=jnp.float32)
    a = jnp.abs(u)
    b = a
    m = None
    for i in range(_K):
        m = jnp.max(b, axis=1, keepdims=True)
        if i < _K - 1:
            b = jnp.where(b >= m, -1.0, b)
    o_ref[...] = jnp.where(a >= m, u, 0.0)


@jax.jit
def kernel(x, z_prev, W, S):
    n, d_in = x.shape
    code = W.shape[0]
    blk = 1024
    grid = n // blk
    return pl.pallas_call(
        _body,
        grid=(grid,),
        in_specs=[
            pl.BlockSpec((blk, d_in), lambda i: (i, 0)),
            pl.BlockSpec((blk, code), lambda i: (i, 0)),
            pl.BlockSpec((code, d_in), lambda i: (0, 0)),
            pl.BlockSpec((code, code), lambda i: (0, 0)),
        ],
        out_specs=pl.BlockSpec((blk, code), lambda i: (i, 0)),
        out_shape=jax.ShapeDtypeStruct((n, code), jnp.float32),
    )(x, z_prev, W, S)
